# Initial kernel scaffold; baseline (speedup 1.0000x reference)
import jax
import jax.numpy as jnp
from jax.experimental import pallas as pl


def kernel(x, edge_index, z, W, b, alpha, bias_edge):
    raise NotImplementedError("write your pallas kernel here")


# speedup vs baseline: 1.5157x; 1.5157x over previous
import jax
import jax.numpy as jnp
from jax.experimental import pallas as pl


def _linear_body(acc_ref, w_ref, b_ref, o_ref):
    o_ref[...] = jnp.dot(acc_ref[...], w_ref[...],
                         preferred_element_type=jnp.float32) + b_ref[...]


def kernel(x, edge_index, z, W, b, alpha, bias_edge):
    N, D = x.shape
    row = edge_index[0]
    col = edge_index[1]
    zn = z * jax.lax.rsqrt(jnp.maximum((z * z).sum(-1, keepdims=True), 1e-18))
    corr = (jnp.take(zn, row, axis=0) * jnp.take(zn, col, axis=0)).sum(-1)
    e = jnp.exp(alpha * corr + bias_edge)
    denom = jnp.zeros((N,), jnp.float32).at[row].add(e)
    msgs = e[:, None] * jnp.take(x, col, axis=0)
    acc = jnp.zeros_like(x).at[row].add(msgs)
    out = acc / (denom[:, None] + 1e-9)

    BN = 400
    return pl.pallas_call(
        _linear_body,
        grid=(N // BN,),
        in_specs=[
            pl.BlockSpec((BN, D), lambda i: (i, 0)),
            pl.BlockSpec((D, D), lambda i: (0, 0)),
            pl.BlockSpec((1, D), lambda i: (0, 0)),
        ],
        out_specs=pl.BlockSpec((BN, D), lambda i: (i, 0)),
        out_shape=jax.ShapeDtypeStruct((N, D), jnp.float32),
    )(out, W.T, b.reshape(1, D))


# speedup vs baseline: 5.3956x; 3.5598x over previous
import functools

import jax
import jax.numpy as jnp
from jax import lax
from jax.experimental import pallas as pl
from jax.experimental.pallas import tpu as pltpu
from jax.experimental.pallas import tpu_sc as plsc


def _norm_body(z_ref, zn_ref):
    zb = z_ref[...]
    s = jnp.sum(zb * zb, axis=1, keepdims=True)
    zn_ref[...] = zb * lax.rsqrt(jnp.maximum(s, 1e-18))


def _final_body(acc_ref, den_ref, wt_ref, b_ref, o_ref):
    a = acc_ref[0] + acc_ref[1]
    dn = den_ref[0, :, 0] + den_ref[1, :, 0]
    o = a / (dn[:, None] + 1e-9)
    o_ref[...] = jnp.dot(o, wt_ref[...],
                         preferred_element_type=jnp.float32) + b_ref[...]


def _make_sc_kernel(N, E, D, Z):
    NC, NS = 2, 16
    NW = NC * NS
    EPT = E // NW
    C = 80
    NCH = EPT // C
    RPT = -(-N // (NS * 8)) * 8
    N2 = RPT * NS
    ZR = 8
    assert EPT * NW == E and NCH * C == EPT
    assert RPT % ZR == 0

    mesh = plsc.VectorSubcoreMesh(core_axis_name="c", subcore_axis_name="s")

    @functools.partial(
        pl.kernel,
        out_type=(jax.ShapeDtypeStruct((NC, N2, D), jnp.float32),
                  jax.ShapeDtypeStruct((NC, N2, 16), jnp.float32)),
        mesh=mesh,
        compiler_params=pltpu.CompilerParams(needs_layout_passes=False,
                                             use_tc_tiling_on_sc=False),
        scratch_types=[
            pltpu.VMEM((C,), jnp.int32),
            pltpu.VMEM((C,), jnp.int32),
            pltpu.VMEM((C, Z), jnp.float32),
            pltpu.VMEM((C, Z), jnp.float32),
            pltpu.VMEM((C, D), jnp.float32),
            pltpu.VMEM((C, 16), jnp.float32),
            pltpu.VMEM((16,), jnp.float32),
            pltpu.VMEM((ZR, D), jnp.float32),
            pltpu.VMEM((ZR, 16), jnp.float32),
            pltpu.VMEM_SHARED((N2, D), jnp.float32),
            pltpu.VMEM_SHARED((N2, 16), jnp.float32),
            pltpu.SemaphoreType.DMA,
            pltpu.SemaphoreType.DMA,
            pltpu.SemaphoreType.DMA,
        ],
    )
    def sc_kernel(zn, x, row, col, par, acc_out, den_out,
                  row_v, col_v, zi_v, zj_v, xj_v, denb_v, par_v,
                  zrow_v, zden_v, acc_sh, den_sh, sem0, sem1, sem2):
        cid = lax.axis_index("c")
        sid = lax.axis_index("s")
        w = sid * NC + cid

        z16 = jnp.zeros((16,), jnp.float32)
        for r in range(ZR):
            for k in range(D // 16):
                zrow_v[r, k * 16:(k + 1) * 16] = z16
            zden_v[r, 0:16] = z16
        for r in range(C):
            denb_v[r, 0:16] = z16
        pltpu.sync_copy(par, par_v)

        def zinit(j, carry):
            base = sid * RPT + j * ZR
            pltpu.sync_copy(zrow_v, acc_sh.at[pl.ds(base, ZR)])
            pltpu.sync_copy(zden_v, den_sh.at[pl.ds(base, ZR)])
            return carry
        lax.fori_loop(0, RPT // ZR, zinit, 0)
        plsc.subcore_barrier()

        lane = lax.iota(jnp.int32, 16)
        parv = par_v[...]
        al = parv[0]
        be = parv[1]

        def chunk(i, carry):
            base = w * EPT + i * C
            pltpu.sync_copy(row.at[pl.ds(base, C)], row_v)
            pltpu.sync_copy(col.at[pl.ds(base, C)], col_v)
            cp0 = pltpu.async_copy(zn.at[row_v], zi_v, sem0)
            cp1 = pltpu.async_copy(zn.at[col_v], zj_v, sem1)
            cp2 = pltpu.async_copy(x.at[col_v], xj_v, sem2)
            cp0.wait()
            cp1.wait()
            cp2.wait()
            for g in range(C // 16):
                eidx = lane + g * 16
                acc16 = jnp.zeros((16,), jnp.float32)
                for d in range(Z):
                    dd = lane * 0 + d
                    a = plsc.load_gather(zi_v, [eidx, dd])
                    bb = plsc.load_gather(zj_v, [eidx, dd])
                    acc16 = acc16 + a * bb
                e16 = jnp.exp(al * acc16 + be)
                plsc.store_scatter(denb_v, [eidx, lane * 0], e16)
                for l in range(16):
                    e = g * 16 + l
                    s = e16[l]
                    for k in range(D // 16):
                        sl = pl.ds(k * 16, 16)
                        xj_v[e, sl] = xj_v[e, sl] * s
            pltpu.sync_copy(xj_v, acc_sh.at[row_v], add=True)
            pltpu.sync_copy(denb_v, den_sh.at[row_v], add=True)
            return carry
        lax.fori_loop(0, NCH, chunk, 0)

        plsc.subcore_barrier()
        base = sid * RPT
        pltpu.sync_copy(acc_sh.at[pl.ds(base, RPT)],
                        acc_out.at[cid].at[pl.ds(base, RPT)])
        pltpu.sync_copy(den_sh.at[pl.ds(base, RPT)],
                        den_out.at[cid].at[pl.ds(base, RPT)])

    return sc_kernel


def kernel(x, edge_index, z, W, b, alpha, bias_edge):
    N, D = x.shape
    Z = z.shape[1]
    E = edge_index.shape[1]
    row = edge_index[0]
    col = edge_index[1]

    BN = 400
    zn = pl.pallas_call(
        _norm_body,
        grid=(N // BN,),
        in_specs=[pl.BlockSpec((BN, Z), lambda i: (i, 0))],
        out_specs=pl.BlockSpec((BN, Z), lambda i: (i, 0)),
        out_shape=jax.ShapeDtypeStruct((N, Z), jnp.float32),
    )(z)

    par = jnp.concatenate([
        jnp.reshape(alpha, (1,)).astype(jnp.float32),
        jnp.reshape(bias_edge, (1,)).astype(jnp.float32),
        jnp.zeros((14,), jnp.float32),
    ])

    acc, den = _make_sc_kernel(N, E, D, Z)(zn, x, row, col, par)

    return pl.pallas_call(
        _final_body,
        grid=(N // BN,),
        in_specs=[
            pl.BlockSpec((2, BN, D), lambda i: (0, i, 0)),
            pl.BlockSpec((2, BN, 16), lambda i: (0, i, 0)),
            pl.BlockSpec((D, D), lambda i: (0, 0)),
            pl.BlockSpec((1, D), lambda i: (0, 0)),
        ],
        out_specs=pl.BlockSpec((BN, D), lambda i: (i, 0)),
        out_shape=jax.ShapeDtypeStruct((N, D), jnp.float32),
    )(acc, den, W.T, b.reshape(1, D))


# speedup vs baseline: 5.5494x; 1.0285x over previous
import functools

import jax
import jax.numpy as jnp
from jax import lax
from jax.experimental import pallas as pl
from jax.experimental.pallas import tpu as pltpu
from jax.experimental.pallas import tpu_sc as plsc


def _norm_body(z_ref, zn_ref):
    zb = z_ref[...]
    s = jnp.sum(zb * zb, axis=1, keepdims=True)
    zn_ref[...] = zb * lax.rsqrt(jnp.maximum(s, 1e-18))


def _final_body(acc_ref, den_ref, wt_ref, b_ref, o_ref):
    a = jnp.concatenate(
        [acc_ref[0, 0] + acc_ref[1, 0], acc_ref[0, 1] + acc_ref[1, 1]],
        axis=1)
    dn = den_ref[0, :, 0] + den_ref[1, :, 0]
    o = a / (dn[:, None] + 1e-9)
    o_ref[...] = jnp.dot(o, wt_ref[...],
                         preferred_element_type=jnp.float32) + b_ref[...]


def _make_sc_kernel(N, E, D, Z):
    NC, NS = 2, 16
    NW = NC * NS
    EPT = E // NW
    C = 80
    NCH = EPT // C
    RPT = -(-N // (NS * 8)) * 8
    N2 = RPT * NS
    H = D // 2
    assert EPT * NW == E and NCH * C == EPT and RPT % 8 == 0

    mesh = plsc.VectorSubcoreMesh(core_axis_name="c", subcore_axis_name="s")

    @functools.partial(
        pl.kernel,
        out_type=(jax.ShapeDtypeStruct((NC, 2, N2, H), jnp.float32),
                  jax.ShapeDtypeStruct((NC, N2, 16), jnp.float32)),
        mesh=mesh,
        compiler_params=pltpu.CompilerParams(needs_layout_passes=False,
                                             use_tc_tiling_on_sc=False),
        scratch_types=[
            [pltpu.VMEM((C,), jnp.int32)] * 2,
            [pltpu.VMEM((C,), jnp.int32)] * 2,
            [pltpu.VMEM((C,), jnp.int32)] * 2,
            [pltpu.VMEM((C, Z), jnp.float32)] * 2,
            [pltpu.VMEM((C, Z), jnp.float32)] * 2,
            [pltpu.VMEM((C, H), jnp.float32)] * 2,
            [pltpu.VMEM((C, H), jnp.float32)] * 2,
            [pltpu.VMEM((C, 16), jnp.float32)] * 2,
            pltpu.VMEM((EPT,), jnp.float32),
            pltpu.VMEM((16,), jnp.float32),
            pltpu.VMEM((8, H), jnp.float32),
            pltpu.VMEM((8, 16), jnp.float32),
            pltpu.VMEM_SHARED((N2, H), jnp.float32),
            pltpu.VMEM_SHARED((N2, 16), jnp.float32),
            [pltpu.SemaphoreType.DMA] * 2,
            [pltpu.SemaphoreType.DMA] * 2,
        ],
    )
    def sc_kernel(zn, xa, xb, row, col, par, acc_out, den_out,
                  row_v, col_v, srow_v, zi_v, zj_v, xj_v, msg_v, denb_v,
                  e_v, par_v, zrow_v, zden_v, acc_sh, den_sh, gsem, ssem):
        cid = lax.axis_index("c")
        sid = lax.axis_index("s")
        w = sid * NC + cid
        lane = lax.iota(jnp.int32, 16)
        z16 = jnp.zeros((16,), jnp.float32)

        def issue_idx(c, b):
            base = w * EPT + c * C
            pltpu.sync_copy(row.at[pl.ds(base, C)], row_v[b])
            pltpu.sync_copy(col.at[pl.ds(base, C)], col_v[b])

        def issue_g1(c, b):
            issue_idx(c, b)
            pltpu.async_copy(zn.at[row_v[b]], zi_v[b], gsem[b])
            pltpu.async_copy(zn.at[col_v[b]], zj_v[b], gsem[b])
            pltpu.async_copy(xa.at[col_v[b]], xj_v[b], gsem[b])

        def wait_g1(b):
            pltpu.make_async_copy(zn.at[row_v[b]], zi_v[b], gsem[b]).wait()
            pltpu.make_async_copy(zn.at[col_v[b]], zj_v[b], gsem[b]).wait()
            pltpu.make_async_copy(xa.at[col_v[b]], xj_v[b], gsem[b]).wait()

        def issue_g2(c, b):
            issue_idx(c, b)
            pltpu.async_copy(xb.at[col_v[b]], xj_v[b], gsem[b])

        def wait_g2(b):
            pltpu.make_async_copy(xb.at[col_v[b]], xj_v[b], gsem[b]).wait()

        def issue_s(b, with_den):
            for k in range(C // 16):
                sl = pl.ds(k * 16, 16)
                srow_v[b][sl] = row_v[b][sl]
            pltpu.async_copy(msg_v[b], acc_sh.at[srow_v[b]], ssem[b],
                             add=True)
            if with_den:
                pltpu.async_copy(denb_v[b], den_sh.at[srow_v[b]], ssem[b],
                                 add=True)

        def wait_s(b, with_den):
            pltpu.make_async_copy(
                msg_v[b], acc_sh.at[srow_v[b]], ssem[b]).wait()
            if with_den:
                pltpu.make_async_copy(
                    denb_v[b], den_sh.at[srow_v[b]], ssem[b]).wait()

        def scale(b, e16, g):
            for l in range(16):
                e = g * 16 + l
                s = e16[l]
                for k in range(H // 16):
                    sl = pl.ds(k * 16, 16)
                    msg_v[b][e, sl] = xj_v[b][e, sl] * s

        def zero_acc(j, carry):
            pltpu.sync_copy(zrow_v, acc_sh.at[pl.ds(sid * RPT + j * 8, 8)])
            return carry

        issue_g1(jnp.int32(0), 0)
        issue_g1(jnp.int32(1), 1)
        for r in range(8):
            for k in range(H // 16):
                zrow_v[r, k * 16:(k + 1) * 16] = z16
            zden_v[r, 0:16] = z16
        for b in range(2):
            for r in range(C):
                denb_v[b][r, 0:16] = z16
        pltpu.sync_copy(par, par_v)
        parv = par_v[...]
        al = parv[0]
        be = parv[1]

        def zinit(j, carry):
            base = sid * RPT + j * 8
            pltpu.sync_copy(zrow_v, acc_sh.at[pl.ds(base, 8)])
            pltpu.sync_copy(zden_v, den_sh.at[pl.ds(base, 8)])
            return carry
        lax.fori_loop(0, RPT // 8, zinit, 0)
        plsc.subcore_barrier()

        def body1(i, carry):
            for b in range(2):
                c = 2 * i + b

                def process(b=b, c=c, i=i):
                    wait_g1(b)
                    lax.cond(i > 0, lambda: wait_s(b, True), lambda: None)
                    for g in range(C // 16):
                        eidx = lane + g * 16
                        acc16 = jnp.zeros((16,), jnp.float32)
                        for d in range(Z):
                            dd = lane * 0 + d
                            a = plsc.load_gather(zi_v[b], [eidx, dd])
                            bb = plsc.load_gather(zj_v[b], [eidx, dd])
                            acc16 = acc16 + a * bb
                        e16 = jnp.exp(al * acc16 + be)
                        e_v[pl.ds(c * C + g * 16, 16)] = e16
                        plsc.store_scatter(denb_v[b], [eidx, lane * 0], e16)
                        scale(b, e16, g)
                    issue_s(b, True)
                    lax.cond(c + 2 < NCH,
                             lambda: issue_g1(c + 2, b), lambda: None)
                lax.cond(c < NCH, process, lambda: None)
            return carry
        lax.fori_loop(0, (NCH + 1) // 2, body1, 0)
        wait_s(0, True)
        wait_s(1, True)
        plsc.subcore_barrier()

        base = sid * RPT
        pltpu.sync_copy(acc_sh.at[pl.ds(base, RPT)],
                        acc_out.at[cid].at[0].at[pl.ds(base, RPT)])
        pltpu.sync_copy(den_sh.at[pl.ds(base, RPT)],
                        den_out.at[cid].at[pl.ds(base, RPT)])
        lax.fori_loop(0, RPT // 8, zero_acc, 0)
        issue_g2(jnp.int32(0), 0)
        issue_g2(jnp.int32(1), 1)
        plsc.subcore_barrier()

        def body2(i, carry):
            for b in range(2):
                c = 2 * i + b

                def process(b=b, c=c, i=i):
                    wait_g2(b)
                    lax.cond(i > 0, lambda: wait_s(b, False), lambda: None)
                    for g in range(C // 16):
                        e16 = e_v[pl.ds(c * C + g * 16, 16)]
                        scale(b, e16, g)
                    issue_s(b, False)
                    lax.cond(c + 2 < NCH,
                             lambda: issue_g2(c + 2, b), lambda: None)
                lax.cond(c < NCH, process, lambda: None)
            return carry
        lax.fori_loop(0, (NCH + 1) // 2, body2, 0)
        wait_s(0, False)
        wait_s(1, False)
        plsc.subcore_barrier()
        pltpu.sync_copy(acc_sh.at[pl.ds(base, RPT)],
                        acc_out.at[cid].at[1].at[pl.ds(base, RPT)])

    return sc_kernel


def kernel(x, edge_index, z, W, b, alpha, bias_edge):
    N, D = x.shape
    Z = z.shape[1]
    E = edge_index.shape[1]
    row = edge_index[0]
    col = edge_index[1]
    H = D // 2

    BN = 400
    zn = pl.pallas_call(
        _norm_body,
        grid=(N // BN,),
        in_specs=[pl.BlockSpec((BN, Z), lambda i: (i, 0))],
        out_specs=pl.BlockSpec((BN, Z), lambda i: (i, 0)),
        out_shape=jax.ShapeDtypeStruct((N, Z), jnp.float32),
    )(z)

    par = jnp.concatenate([
        jnp.reshape(alpha, (1,)).astype(jnp.float32),
        jnp.reshape(bias_edge, (1,)).astype(jnp.float32),
        jnp.zeros((14,), jnp.float32),
    ])

    xa = x[:, :H]
    xb = x[:, H:]
    acc, den = _make_sc_kernel(N, E, D, Z)(zn, xa, xb, row, col, par)

    return pl.pallas_call(
        _final_body,
        grid=(N // BN,),
        in_specs=[
            pl.BlockSpec((2, 2, BN, H), lambda i: (0, 0, i, 0)),
            pl.BlockSpec((2, BN, 16), lambda i: (0, i, 0)),
            pl.BlockSpec((D, D), lambda i: (0, 0)),
            pl.BlockSpec((1, D), lambda i: (0, 0)),
        ],
        out_specs=pl.BlockSpec((BN, D), lambda i: (i, 0)),
        out_shape=jax.ShapeDtypeStruct((N, D), jnp.float32),
    )(acc, den, W.T, b.reshape(1, D))


# speedup vs baseline: 5.5961x; 1.0084x over previous
import functools

import jax
import jax.numpy as jnp
from jax import lax
from jax.experimental import pallas as pl
from jax.experimental.pallas import tpu as pltpu
from jax.experimental.pallas import tpu_sc as plsc


def _norm_body(z_ref, zn_ref):
    zb = z_ref[...]
    s = jnp.sum(zb * zb, axis=1, keepdims=True)
    zn_ref[...] = zb * lax.rsqrt(jnp.maximum(s, 1e-18))


def _final_body(acc_ref, den_ref, wt_ref, b_ref, o_ref):
    a = jnp.concatenate(
        [acc_ref[0, 0] + acc_ref[1, 0], acc_ref[0, 1] + acc_ref[1, 1]],
        axis=1)
    dn = den_ref[0, :, 0] + den_ref[1, :, 0]
    o = a / (dn[:, None] + 1e-9)
    o_ref[...] = jnp.dot(o, wt_ref[...],
                         preferred_element_type=jnp.float32) + b_ref[...]


def _make_sc_kernel(N, E, D, Z):
    NC, NS = 2, 16
    NW = NC * NS
    EPT = E // NW
    C = 80
    NCH = EPT // C
    RPT = -(-N // (NS * 8)) * 8
    N2 = RPT * NS
    H = D // 2
    assert EPT * NW == E and NCH * C == EPT and RPT % 8 == 0

    mesh = plsc.VectorSubcoreMesh(core_axis_name="c", subcore_axis_name="s")

    @functools.partial(
        pl.kernel,
        out_type=(jax.ShapeDtypeStruct((NC, 2, N2, H), jnp.float32),
                  jax.ShapeDtypeStruct((NC, N2, 16), jnp.float32)),
        mesh=mesh,
        compiler_params=pltpu.CompilerParams(needs_layout_passes=False,
                                             use_tc_tiling_on_sc=False),
        scratch_types=[
            [pltpu.VMEM((C,), jnp.int32)] * 2,
            [pltpu.VMEM((C,), jnp.int32)] * 2,
            [pltpu.VMEM((C,), jnp.int32)] * 2,
            [pltpu.VMEM((C, Z), jnp.float32)] * 2,
            [pltpu.VMEM((C, Z), jnp.float32)] * 2,
            [pltpu.VMEM((C, H), jnp.float32)] * 2,
            [pltpu.VMEM((C, H), jnp.float32)] * 2,
            [pltpu.VMEM((C, 16), jnp.float32)] * 2,
            pltpu.VMEM((EPT,), jnp.float32),
            pltpu.VMEM((16,), jnp.float32),
            pltpu.VMEM((8, H), jnp.float32),
            pltpu.VMEM((8, 16), jnp.float32),
            pltpu.VMEM_SHARED((N2, H), jnp.float32),
            pltpu.VMEM_SHARED((N2, 16), jnp.float32),
            [pltpu.SemaphoreType.DMA] * 2,
            [pltpu.SemaphoreType.DMA] * 2,
        ],
    )
    def sc_kernel(zn, xa, xb, row, col, par, acc_out, den_out,
                  row_v, col_v, srow_v, zi_v, zj_v, xj_v, msg_v, denb_v,
                  e_v, par_v, zrow_v, zden_v, acc_sh, den_sh, gsem, ssem):
        cid = lax.axis_index("c")
        sid = lax.axis_index("s")
        w = sid * NC + cid
        lane = lax.iota(jnp.int32, 16)
        z16 = jnp.zeros((16,), jnp.float32)

        def issue_idx(c, b):
            base = w * EPT + c * C
            pltpu.sync_copy(row.at[pl.ds(base, C)], row_v[b])
            pltpu.sync_copy(col.at[pl.ds(base, C)], col_v[b])

        def issue_g1(c, b):
            issue_idx(c, b)
            pltpu.async_copy(zn.at[row_v[b]], zi_v[b], gsem[b])
            pltpu.async_copy(zn.at[col_v[b]], zj_v[b], gsem[b])
            pltpu.async_copy(xa.at[col_v[b]], xj_v[b], gsem[b])

        def wait_g1(b):
            pltpu.make_async_copy(zn.at[row_v[b]], zi_v[b], gsem[b]).wait()
            pltpu.make_async_copy(zn.at[col_v[b]], zj_v[b], gsem[b]).wait()
            pltpu.make_async_copy(xa.at[col_v[b]], xj_v[b], gsem[b]).wait()

        def issue_g2(c, b):
            issue_idx(c, b)
            pltpu.async_copy(xb.at[col_v[b]], xj_v[b], gsem[b])

        def wait_g2(b):
            pltpu.make_async_copy(xb.at[col_v[b]], xj_v[b], gsem[b]).wait()

        def issue_s(b, with_den):
            for k in range(C // 16):
                sl = pl.ds(k * 16, 16)
                srow_v[b][sl] = row_v[b][sl]
            pltpu.async_copy(msg_v[b], acc_sh.at[srow_v[b]], ssem[b],
                             add=True)
            if with_den:
                pltpu.async_copy(denb_v[b], den_sh.at[srow_v[b]], ssem[b],
                                 add=True)

        def wait_s(b, with_den):
            pltpu.make_async_copy(
                msg_v[b], acc_sh.at[srow_v[b]], ssem[b]).wait()
            if with_den:
                pltpu.make_async_copy(
                    denb_v[b], den_sh.at[srow_v[b]], ssem[b]).wait()

        def scale(b, e16, g):
            for l in range(16):
                e = g * 16 + l
                s = e16[l]
                for k in range(H // 16):
                    sl = pl.ds(k * 16, 16)
                    msg_v[b][e, sl] = xj_v[b][e, sl] * s

        def zero_acc(j, carry):
            pltpu.sync_copy(zrow_v, acc_sh.at[pl.ds(sid * RPT + j * 8, 8)])
            return carry

        issue_g1(jnp.int32(0), 0)
        issue_g1(jnp.int32(1), 1)
        for r in range(8):
            for k in range(H // 16):
                zrow_v[r, k * 16:(k + 1) * 16] = z16
            zden_v[r, 0:16] = z16
        for b in range(2):
            for r in range(C):
                denb_v[b][r, 0:16] = z16
        pltpu.sync_copy(par, par_v)
        parv = par_v[...]
        al = parv[0]
        be = parv[1]

        def zinit(j, carry):
            base = sid * RPT + j * 8
            pltpu.sync_copy(zrow_v, acc_sh.at[pl.ds(base, 8)])
            pltpu.sync_copy(zden_v, den_sh.at[pl.ds(base, 8)])
            return carry
        lax.fori_loop(0, RPT // 8, zinit, 0)
        plsc.subcore_barrier()

        def body1(i, carry):
            for b in range(2):
                c = 2 * i + b

                def process(b=b, c=c, i=i):
                    wait_g1(b)
                    lax.cond(i > 0, lambda: wait_s(b, True), lambda: None)
                    for g in range(C // 16):
                        eidx = lane + g * 16
                        accs = [jnp.zeros((16,), jnp.float32)
                                for _ in range(8)]
                        for d in range(Z):
                            dd = lane * 0 + d
                            a = plsc.load_gather(zi_v[b], [eidx, dd])
                            bb = plsc.load_gather(zj_v[b], [eidx, dd])
                            accs[d % 8] = accs[d % 8] + a * bb
                        acc16 = (((accs[0] + accs[1]) + (accs[2] + accs[3]))
                                 + ((accs[4] + accs[5]) + (accs[6] + accs[7])))
                        e16 = jnp.exp(al * acc16 + be)
                        e_v[pl.ds(c * C + g * 16, 16)] = e16
                        plsc.store_scatter(denb_v[b], [eidx, lane * 0], e16)
                        scale(b, e16, g)
                    issue_s(b, True)
                    lax.cond(c + 2 < NCH,
                             lambda: issue_g1(c + 2, b), lambda: None)
                lax.cond(c < NCH, process, lambda: None)
            return carry
        lax.fori_loop(0, (NCH + 1) // 2, body1, 0)
        wait_s(0, True)
        wait_s(1, True)
        plsc.subcore_barrier()

        base = sid * RPT
        pltpu.sync_copy(acc_sh.at[pl.ds(base, RPT)],
                        acc_out.at[cid].at[0].at[pl.ds(base, RPT)])
        pltpu.sync_copy(den_sh.at[pl.ds(base, RPT)],
                        den_out.at[cid].at[pl.ds(base, RPT)])
        lax.fori_loop(0, RPT // 8, zero_acc, 0)
        issue_g2(jnp.int32(0), 0)
        issue_g2(jnp.int32(1), 1)
        plsc.subcore_barrier()

        def body2(i, carry):
            for b in range(2):
                c = 2 * i + b

                def process(b=b, c=c, i=i):
                    wait_g2(b)
                    lax.cond(i > 0, lambda: wait_s(b, False), lambda: None)
                    for g in range(C // 16):
                        e16 = e_v[pl.ds(c * C + g * 16, 16)]
                        scale(b, e16, g)
                    issue_s(b, False)
                    lax.cond(c + 2 < NCH,
                             lambda: issue_g2(c + 2, b), lambda: None)
                lax.cond(c < NCH, process, lambda: None)
            return carry
        lax.fori_loop(0, (NCH + 1) // 2, body2, 0)
        wait_s(0, False)
        wait_s(1, False)
        plsc.subcore_barrier()
        pltpu.sync_copy(acc_sh.at[pl.ds(base, RPT)],
                        acc_out.at[cid].at[1].at[pl.ds(base, RPT)])

    return sc_kernel


def kernel(x, edge_index, z, W, b, alpha, bias_edge):
    N, D = x.shape
    Z = z.shape[1]
    E = edge_index.shape[1]
    row = edge_index[0]
    col = edge_index[1]
    H = D // 2

    BN = 400
    zn = pl.pallas_call(
        _norm_body,
        grid=(N // BN,),
        in_specs=[pl.BlockSpec((BN, Z), lambda i: (i, 0))],
        out_specs=pl.BlockSpec((BN, Z), lambda i: (i, 0)),
        out_shape=jax.ShapeDtypeStruct((N, Z), jnp.float32),
    )(z)

    par = jnp.concatenate([
        jnp.reshape(alpha, (1,)).astype(jnp.float32),
        jnp.reshape(bias_edge, (1,)).astype(jnp.float32),
        jnp.zeros((14,), jnp.float32),
    ])

    xa = x[:, :H]
    xb = x[:, H:]
    acc, den = _make_sc_kernel(N, E, D, Z)(zn, xa, xb, row, col, par)

    return pl.pallas_call(
        _final_body,
        grid=(N // BN,),
        in_specs=[
            pl.BlockSpec((2, 2, BN, H), lambda i: (0, 0, i, 0)),
            pl.BlockSpec((2, BN, 16), lambda i: (0, i, 0)),
            pl.BlockSpec((D, D), lambda i: (0, 0)),
            pl.BlockSpec((1, D), lambda i: (0, 0)),
        ],
        out_specs=pl.BlockSpec((BN, D), lambda i: (i, 0)),
        out_shape=jax.ShapeDtypeStruct((N, D), jnp.float32),
    )(acc, den, W.T, b.reshape(1, D))


# speedup vs baseline: 9.5358x; 1.7040x over previous
import functools

import jax
import jax.numpy as jnp
import numpy as np
from jax import lax
from jax.experimental import pallas as pl
from jax.experimental.pallas import tpu as pltpu
from jax.experimental.pallas import tpu_sc as plsc


def _norm_body(z_ref, zn_ref):
    zb = z_ref[...]
    s = jnp.sum(zb * zb, axis=1, keepdims=True)
    zn_ref[...] = (zb * lax.rsqrt(jnp.maximum(s, 1e-18))).astype(jnp.bfloat16)


def _final_body(acc_ref, den_ref, wt_ref, b_ref, o_ref):
    a = jnp.concatenate(
        [acc_ref[0, 0] + acc_ref[1, 0], acc_ref[0, 1] + acc_ref[1, 1]],
        axis=1)
    dn = den_ref[0, :, 0] + den_ref[1, :, 0]
    o = a / (dn[:, None] + 1e-9)
    o_ref[...] = jnp.dot(o, wt_ref[...],
                         preferred_element_type=jnp.float32) + b_ref[...]


def _make_sc_kernel(N, E, D, Z):
    NC, NS = 2, 16
    NW = NC * NS
    EPT = E // NW
    C = 80
    NCH = EPT // C
    RPT = -(-N // (NS * 8)) * 8
    N2 = RPT * NS
    H = D // 2
    ZP = Z // 2
    HP = H // 2
    assert EPT * NW == E and NCH * C == EPT and RPT % 8 == 0

    mesh = plsc.VectorSubcoreMesh(core_axis_name="c", subcore_axis_name="s")

    @functools.partial(
        pl.kernel,
        out_type=(jax.ShapeDtypeStruct((NC, 2, N2, H), jnp.float32),
                  jax.ShapeDtypeStruct((NC, N2, 16), jnp.float32)),
        mesh=mesh,
        compiler_params=pltpu.CompilerParams(needs_layout_passes=False,
                                             use_tc_tiling_on_sc=False),
        scratch_types=[
            pltpu.VMEM((NCH, C), jnp.int32),
            pltpu.VMEM((NCH, C), jnp.int32),
            [pltpu.VMEM((C, ZP), jnp.int32)] * 2,
            [pltpu.VMEM((C, ZP), jnp.int32)] * 2,
            [pltpu.VMEM((C, HP), jnp.int32)] * 2,
            [pltpu.VMEM((C, H), jnp.float32)] * 2,
            [pltpu.VMEM((C, 16), jnp.float32)] * 2,
            pltpu.VMEM((EPT,), jnp.float32),
            pltpu.VMEM((16,), jnp.float32),
            pltpu.VMEM((8, H), jnp.float32),
            pltpu.VMEM((8, 16), jnp.float32),
            pltpu.VMEM_SHARED((N2, H), jnp.float32),
            pltpu.VMEM_SHARED((N2, 16), jnp.float32),
            [pltpu.SemaphoreType.DMA] * 2,
            [pltpu.SemaphoreType.DMA] * 2,
        ],
    )
    def sc_kernel(znp, xap, xbp, row2, col2, par, acc_out, den_out,
                  row_a, col_a, zi_v, zj_v, xj_v, msg_v, denb_v,
                  e_v, par_v, zrow_v, zden_v, acc_sh, den_sh, gsem, ssem):
        cid = lax.axis_index("c")
        sid = lax.axis_index("s")
        w = sid * NC + cid
        lane = lax.iota(jnp.int32, 16)
        z16 = jnp.zeros((16,), jnp.float32)

        def issue_g1(c, b):
            pltpu.async_copy(znp.at[row_a.at[c]], zi_v[b], gsem[b])
            pltpu.async_copy(znp.at[col_a.at[c]], zj_v[b], gsem[b])
            pltpu.async_copy(xap.at[col_a.at[c]], xj_v[b], gsem[b])

        def wait_g1(c, b):
            pltpu.make_async_copy(znp.at[row_a.at[c]], zi_v[b],
                                  gsem[b]).wait()
            pltpu.make_async_copy(znp.at[col_a.at[c]], zj_v[b],
                                  gsem[b]).wait()
            pltpu.make_async_copy(xap.at[col_a.at[c]], xj_v[b],
                                  gsem[b]).wait()

        def issue_g2(c, b):
            pltpu.async_copy(xbp.at[col_a.at[c]], xj_v[b], gsem[b])

        def wait_g2(c, b):
            pltpu.make_async_copy(xbp.at[col_a.at[c]], xj_v[b],
                                  gsem[b]).wait()

        def issue_s(c, b, with_den):
            pltpu.async_copy(msg_v[b], acc_sh.at[row_a.at[c]], ssem[b],
                             add=True)
            if with_den:
                pltpu.async_copy(denb_v[b], den_sh.at[row_a.at[c]], ssem[b],
                                 add=True)

        def wait_s(c, b, with_den):
            pltpu.make_async_copy(
                msg_v[b], acc_sh.at[row_a.at[c]], ssem[b]).wait()
            if with_den:
                pltpu.make_async_copy(
                    denb_v[b], den_sh.at[row_a.at[c]], ssem[b]).wait()

        def scale(b, e16, g):
            for l in range(16):
                e = g * 16 + l
                s = e16[l]
                for k in range(HP // 16):
                    pk = plsc.bitcast(xj_v[b][e, pl.ds(k * 16, 16)],
                                      jnp.bfloat16)
                    va, vb = plsc.unpack(pk,
                                         format=plsc.PackFormat.INTERLEAVED)
                    msg_v[b][e, pl.ds(k * 32, 16)] = va * s
                    msg_v[b][e, pl.ds(k * 32 + 16, 16)] = vb * s

        def zero_acc(j, carry):
            pltpu.sync_copy(zrow_v, acc_sh.at[pl.ds(sid * RPT + j * 8, 8)])
            return carry

        pltpu.sync_copy(row2.at[w], row_a)
        pltpu.sync_copy(col2.at[w], col_a)
        issue_g1(jnp.int32(0), 0)
        issue_g1(jnp.int32(1), 1)
        for r in range(8):
            for k in range(H // 16):
                zrow_v[r, k * 16:(k + 1) * 16] = z16
            zden_v[r, 0:16] = z16
        for b in range(2):
            for r in range(C):
                denb_v[b][r, 0:16] = z16
        pltpu.sync_copy(par, par_v)
        parv = par_v[...]
        al = parv[0]
        be = parv[1]

        def zinit(j, carry):
            base = sid * RPT + j * 8
            pltpu.sync_copy(zrow_v, acc_sh.at[pl.ds(base, 8)])
            pltpu.sync_copy(zden_v, den_sh.at[pl.ds(base, 8)])
            return carry
        lax.fori_loop(0, RPT // 8, zinit, 0)
        plsc.subcore_barrier()

        def body1(i, carry):
            for b in range(2):
                c = 2 * i + b

                def process(b=b, c=c, i=i):
                    wait_g1(c, b)
                    lax.cond(i > 0,
                             lambda: wait_s(c - 2, b, True), lambda: None)
                    for g in range(C // 16):
                        eidx = lane + g * 16
                        accs = [jnp.zeros((16,), jnp.float32)
                                for _ in range(8)]
                        for d in range(ZP):
                            dd = lane * 0 + d
                            pa = plsc.bitcast(
                                plsc.load_gather(zi_v[b], [eidx, dd]),
                                jnp.bfloat16)
                            pb = plsc.bitcast(
                                plsc.load_gather(zj_v[b], [eidx, dd]),
                                jnp.bfloat16)
                            ia, ib = plsc.unpack(
                                pa, format=plsc.PackFormat.INTERLEAVED)
                            ja, jb = plsc.unpack(
                                pb, format=plsc.PackFormat.INTERLEAVED)
                            k = d % 4
                            accs[2 * k] = accs[2 * k] + ia * ja
                            accs[2 * k + 1] = accs[2 * k + 1] + ib * jb
                        acc16 = (((accs[0] + accs[1]) + (accs[2] + accs[3]))
                                 + ((accs[4] + accs[5])
                                    + (accs[6] + accs[7])))
                        e16 = jnp.exp(al * acc16 + be)
                        e_v[pl.ds(c * C + g * 16, 16)] = e16
                        plsc.store_scatter(denb_v[b], [eidx, lane * 0], e16)
                        scale(b, e16, g)
                    issue_s(c, b, True)
                    lax.cond(c + 2 < NCH,
                             lambda: issue_g1(c + 2, b), lambda: None)
                lax.cond(c < NCH, process, lambda: None)
            return carry
        lax.fori_loop(0, (NCH + 1) // 2, body1, 0)
        wait_s(jnp.int32(NCH - 2), 0, True)
        wait_s(jnp.int32(NCH - 2), 1, True)
        plsc.subcore_barrier()

        base = sid * RPT
        pltpu.sync_copy(acc_sh.at[pl.ds(base, RPT)],
                        acc_out.at[cid].at[0].at[pl.ds(base, RPT)])
        pltpu.sync_copy(den_sh.at[pl.ds(base, RPT)],
                        den_out.at[cid].at[pl.ds(base, RPT)])
        lax.fori_loop(0, RPT // 8, zero_acc, 0)
        issue_g2(jnp.int32(0), 0)
        issue_g2(jnp.int32(1), 1)
        plsc.subcore_barrier()

        def body2(i, carry):
            for b in range(2):
                c = 2 * i + b

                def process(b=b, c=c, i=i):
                    wait_g2(c, b)
                    lax.cond(i > 0,
                             lambda: wait_s(c - 2, b, False), lambda: None)
                    for g in range(C // 16):
                        e16 = e_v[pl.ds(c * C + g * 16, 16)]
                        scale(b, e16, g)
                    issue_s(c, b, False)
                    lax.cond(c + 2 < NCH,
                             lambda: issue_g2(c + 2, b), lambda: None)
                lax.cond(c < NCH, process, lambda: None)
            return carry
        lax.fori_loop(0, (NCH + 1) // 2, body2, 0)
        wait_s(jnp.int32(NCH - 2), 0, False)
        wait_s(jnp.int32(NCH - 2), 1, False)
        plsc.subcore_barrier()
        pltpu.sync_copy(acc_sh.at[pl.ds(base, RPT)],
                        acc_out.at[cid].at[1].at[pl.ds(base, RPT)])

    return sc_kernel


def _pack_rows(a):
    n, m = a.shape
    return lax.bitcast_convert_type(
        a.astype(jnp.bfloat16).reshape(n, m // 2, 2), jnp.int32)


def kernel(x, edge_index, z, W, b, alpha, bias_edge):
    N, D = x.shape
    Z = z.shape[1]
    E = edge_index.shape[1]
    NW = 32
    H = D // 2
    row = edge_index[0]
    col = edge_index[1]

    BN = 400
    znp = pl.pallas_call(
        _norm_body,
        grid=(N // BN,),
        in_specs=[pl.BlockSpec((BN, Z), lambda i: (i, 0))],
        out_specs=pl.BlockSpec((BN, Z), lambda i: (i, 0)),
        out_shape=jax.ShapeDtypeStruct((N, Z), jnp.bfloat16),
    )(z)
    znp = lax.bitcast_convert_type(znp.reshape(N, Z // 2, 2), jnp.int32)

    par = jnp.concatenate([
        jnp.reshape(alpha, (1,)).astype(jnp.float32),
        jnp.reshape(bias_edge, (1,)).astype(jnp.float32),
        jnp.zeros((14,), jnp.float32),
    ])

    xap = _pack_rows(x[:, :H])
    xbp = _pack_rows(x[:, H:])
    row2 = row.reshape(NW, -1, 80)
    col2 = col.reshape(NW, -1, 80)
    acc, den = _make_sc_kernel(N, E, D, Z)(znp, xap, xbp, row2, col2, par)

    perm = np.concatenate(
        [np.concatenate([np.arange(s, s + 32, 2),
                         np.arange(s + 1, s + 32, 2)]) for s in range(0, D, 32)])
    wt_perm = W.T[perm, :]

    return pl.pallas_call(
        _final_body,
        grid=(N // BN,),
        in_specs=[
            pl.BlockSpec((2, 2, BN, H), lambda i: (0, 0, i, 0)),
            pl.BlockSpec((2, BN, 16), lambda i: (0, i, 0)),
            pl.BlockSpec((D, D), lambda i: (0, 0)),
            pl.BlockSpec((1, D), lambda i: (0, 0)),
        ],
        out_specs=pl.BlockSpec((BN, D), lambda i: (i, 0)),
        out_shape=jax.ShapeDtypeStruct((N, D), jnp.float32),
    )(acc, den, wt_perm, b.reshape(1, D))
